# baseline (speedup 1.0000x reference)
import functools

import jax
import jax.numpy as jnp
from jax import lax
from jax.experimental import pallas as pl
from jax.experimental.pallas import tpu as pltpu
from jax.experimental.pallas import tpu_sc as plsc

PH = 16
DH = 1024
NROWS = PH * PH
D = 2 * DH
NC = 2
NS = 16
RPW = NROWS // (NC * NS)

_mesh = plsc.VectorSubcoreMesh(core_axis_name="c", subcore_axis_name="s")


@functools.partial(
    pl.kernel,
    mesh=_mesh,
    out_type=jax.ShapeDtypeStruct((NROWS, D), jnp.float32),
    scratch_types=[
        pltpu.SemaphoreType.DMA,
    ],
)
def _spatial_pos_enc(row_hbm, col_hbm, out_hbm, sem):
    wid = lax.axis_index("s") * NC + lax.axis_index("c")
    i = wid // 2
    j0 = (wid % 2) * RPW
    base = wid * RPW

    copies = [
        pltpu.async_copy(
            col_hbm.at[pl.ds(j0, RPW)],
            out_hbm.at[pl.ds(base, RPW), pl.ds(DH, DH)],
            sem,
        )
    ]
    for t in range(RPW):
        copies.append(
            pltpu.async_copy(
                row_hbm.at[pl.ds(i, 1)],
                out_hbm.at[pl.ds(base + t, 1), pl.ds(0, DH)],
                sem,
            )
        )
    for c in copies:
        c.wait()


def kernel(row_embed, col_embed):
    return _spatial_pos_enc(row_embed, col_embed)


# speedup vs baseline: 3.5908x; 3.5908x over previous
import functools

import jax
import jax.numpy as jnp
from jax import lax
from jax.experimental import pallas as pl
from jax.experimental.pallas import tpu as pltpu
from jax.experimental.pallas import tpu_sc as plsc

PH = 16
DH = 1024
NROWS = PH * PH
D = 2 * DH
NC = 2
NS = 16
RPW = NROWS // (NC * NS)

_mesh = plsc.VectorSubcoreMesh(core_axis_name="c", subcore_axis_name="s")


@functools.partial(
    pl.kernel,
    mesh=_mesh,
    out_type=jax.ShapeDtypeStruct((NROWS, D), jnp.float32),
    scratch_types=[
        pltpu.VMEM((1, DH), jnp.float32),
        pltpu.VMEM((RPW, DH), jnp.float32),
        pltpu.SemaphoreType.DMA,
        pltpu.SemaphoreType.DMA,
        pltpu.SemaphoreType.DMA,
    ],
)
def _spatial_pos_enc(row_hbm, col_hbm, out_hbm, r_v, c_v, sem_r, sem_c, sem_o):
    wid = lax.axis_index("s") * NC + lax.axis_index("c")
    i = wid // 2
    j0 = (wid % 2) * RPW
    base = wid * RPW

    in_r = pltpu.async_copy(row_hbm.at[pl.ds(i, 1)], r_v, sem_r)
    in_c = pltpu.async_copy(col_hbm.at[pl.ds(j0, RPW)], c_v, sem_c)

    in_c.wait()
    outs = [
        pltpu.async_copy(
            c_v, out_hbm.at[pl.ds(base, RPW), pl.ds(DH, DH)], sem_o
        )
    ]
    in_r.wait()
    for t in range(RPW):
        outs.append(
            pltpu.async_copy(
                r_v, out_hbm.at[pl.ds(base + t, 1), pl.ds(0, DH)], sem_o
            )
        )
    for h in outs:
        h.wait()


def kernel(row_embed, col_embed):
    return _spatial_pos_enc(row_embed, col_embed)


# speedup vs baseline: 4.1002x; 1.1418x over previous
import functools

import jax
import jax.numpy as jnp
from jax import lax
from jax.experimental import pallas as pl
from jax.experimental.pallas import tpu as pltpu
from jax.experimental.pallas import tpu_sc as plsc

PH = 16
DH = 1024
NROWS = PH * PH
D = 2 * DH
NC = 2
BLK = 8
NBLK = NROWS // (NC * BLK)

_mesh = plsc.ScalarSubcoreMesh(axis_name="c", num_cores=NC)


@functools.partial(
    pl.kernel,
    mesh=_mesh,
    out_type=jax.ShapeDtypeStruct((NROWS, D), jnp.float32),
    scratch_types=[
        pltpu.MemorySpace.VMEM_SHARED((PH, DH), jnp.float32),
        pltpu.MemorySpace.VMEM_SHARED((PH, DH), jnp.float32),
        pltpu.SemaphoreType.DMA,
        pltpu.SemaphoreType.DMA,
    ],
)
def _spatial_pos_enc(row_hbm, col_hbm, out_hbm, row_sp, col_sp, sem_i, sem_o):
    half = lax.axis_index("c")
    r0 = half * (NROWS // NC)

    in_r = pltpu.async_copy(row_hbm, row_sp, sem_i)
    in_c = pltpu.async_copy(col_hbm, col_sp, sem_i)
    in_r.wait()
    in_c.wait()

    outs = []
    for blk in range(NBLK):
        base = r0 + blk * BLK
        outs.append(
            pltpu.async_copy(
                col_sp.at[pl.ds((blk * BLK) % PH, BLK)],
                out_hbm.at[pl.ds(base, BLK), pl.ds(DH, DH)],
                sem_o,
            )
        )
        for t in range(BLK):
            i = (base + t) // PH
            outs.append(
                pltpu.async_copy(
                    row_sp.at[pl.ds(i, 1)],
                    out_hbm.at[pl.ds(base + t, 1), pl.ds(0, DH)],
                    sem_o,
                )
            )
    for h in outs:
        h.wait()


def kernel(row_embed, col_embed):
    return _spatial_pos_enc(row_embed, col_embed)


# speedup vs baseline: 4.1709x; 1.0173x over previous
import functools

import jax
import jax.numpy as jnp
from jax import lax
from jax.experimental import pallas as pl
from jax.experimental.pallas import tpu as pltpu
from jax.experimental.pallas import tpu_sc as plsc

PH = 16
DH = 1024
NROWS = PH * PH
D = 2 * DH
NC = 2
HALF = NROWS // NC
NGRP = HALF // PH

_mesh = plsc.ScalarSubcoreMesh(axis_name="c", num_cores=NC)


@functools.partial(
    pl.kernel,
    mesh=_mesh,
    out_type=jax.ShapeDtypeStruct((NROWS, D), jnp.float32),
    scratch_types=[
        pltpu.MemorySpace.VMEM_SHARED((PH, DH), jnp.float32),
        pltpu.MemorySpace.VMEM_SHARED((PH, DH), jnp.float32),
        pltpu.MemorySpace.VMEM_SHARED((HALF, D), jnp.float32),
        pltpu.SemaphoreType.DMA,
        pltpu.SemaphoreType.DMA,
    ],
)
def _spatial_pos_enc(
    row_hbm, col_hbm, out_hbm, row_sp, col_sp, drain_sp, sem_i, sem_o
):
    half = lax.axis_index("c")
    r0 = half * HALF
    i0 = half * NGRP

    in_r = pltpu.async_copy(row_hbm, row_sp, sem_i)
    in_c = pltpu.async_copy(col_hbm, col_sp, sem_i)
    in_r.wait()
    in_c.wait()

    def group(g, carry):
        gbase = r0 + g * PH
        i = i0 + g
        pltpu.async_copy(
            col_sp, out_hbm.at[pl.ds(gbase, PH), pl.ds(DH, DH)], sem_o
        )

        def row(t, c2):
            pltpu.async_copy(
                row_sp.at[pl.ds(i, 1)],
                out_hbm.at[pl.ds(gbase + t, 1), pl.ds(0, DH)],
                sem_o,
            )
            return c2

        return lax.fori_loop(0, PH, row, carry)

    lax.fori_loop(0, NGRP, group, 0)

    pltpu.make_async_copy(out_hbm.at[pl.ds(r0, HALF)], drain_sp, sem_o).wait()


def kernel(row_embed, col_embed):
    return _spatial_pos_enc(row_embed, col_embed)


# speedup vs baseline: 4.2468x; 1.0182x over previous
import functools

import jax
import jax.numpy as jnp
from jax import lax
from jax.experimental import pallas as pl
from jax.experimental.pallas import tpu as pltpu
from jax.experimental.pallas import tpu_sc as plsc

PH = 16
DH = 1024
NROWS = PH * PH
D = 2 * DH
NC = 2
HALF = NROWS // NC
NGRP = HALF // PH

_mesh = plsc.ScalarSubcoreMesh(axis_name="c", num_cores=NC)


@functools.partial(
    pl.kernel,
    mesh=_mesh,
    out_type=jax.ShapeDtypeStruct((NROWS, D), jnp.float32),
    scratch_types=[
        pltpu.MemorySpace.VMEM_SHARED((PH, DH), jnp.float32),
        pltpu.MemorySpace.VMEM_SHARED((PH, DH), jnp.float32),
        pltpu.MemorySpace.VMEM_SHARED((HALF, D), jnp.float32),
        pltpu.SemaphoreType.DMA,
        pltpu.SemaphoreType.DMA,
    ],
)
def _spatial_pos_enc(
    row_hbm, col_hbm, out_hbm, row_sp, col_sp, drain_sp, sem_i, sem_o
):
    half = lax.axis_index("c")
    r0 = half * HALF
    i0 = half * NGRP

    in_c = pltpu.async_copy(col_hbm, col_sp, sem_i)
    in_r = pltpu.async_copy(row_hbm, row_sp, sem_i)

    in_c.wait()

    def colg(g, carry):
        pltpu.async_copy(
            col_sp, out_hbm.at[pl.ds(r0 + g * PH, PH), pl.ds(DH, DH)], sem_o
        )
        return carry

    lax.fori_loop(0, NGRP, colg, 0)

    in_r.wait()

    def rowt(t, carry):
        pltpu.async_copy(
            row_sp.at[pl.ds(i0 + t // PH, 1)],
            out_hbm.at[pl.ds(r0 + t, 1), pl.ds(0, DH)],
            sem_o,
        )
        return carry

    lax.fori_loop(0, HALF, rowt, 0)

    pltpu.make_async_copy(out_hbm.at[pl.ds(r0, HALF)], drain_sp, sem_o).wait()


def kernel(row_embed, col_embed):
    return _spatial_pos_enc(row_embed, col_embed)
